# baseline (speedup 1.0000x reference)
import functools

import jax
import jax.numpy as jnp
from jax import lax
from jax.experimental import pallas as pl
from jax.experimental.pallas import tpu as pltpu
from jax.experimental.pallas import tpu_sc as plsc

N = 10000
D = 128
E = 320000

NC = 2
NS = 16
NW = NC * NS

K = 64
NBUF = 5
EPT = 10240
E_PAD = NW * EPT
CHUNKS = EPT // K
GROUPS = CHUNKS // NBUF
N_PAD = 10112
ROWS_PT = N_PAD // NS

_MM_BLOCK = 1000


def _mm_body(x_ref, w_ref, o_ref):
    o_ref[...] = jnp.dot(x_ref[...], w_ref[...],
                         preferred_element_type=jnp.float32)


def _tc_matmul(x, w):
    return pl.pallas_call(
        _mm_body,
        grid=(N // _MM_BLOCK,),
        in_specs=[
            pl.BlockSpec((_MM_BLOCK, D), lambda i: (i, 0)),
            pl.BlockSpec((D, D), lambda i: (0, 0)),
        ],
        out_specs=pl.BlockSpec((_MM_BLOCK, D), lambda i: (i, 0)),
        out_shape=jax.ShapeDtypeStruct((N, D), jnp.float32),
    )(x, w)


def _comb_relu_mm_body(p_ref0, p_ref1, b_ref, w_ref, o_ref):
    h = jnp.maximum(p_ref0[0] + p_ref1[0] + b_ref[...], 0.0)
    o_ref[...] = jnp.dot(h, w_ref[...], preferred_element_type=jnp.float32)


def _tc_combine_relu_matmul(p, b, w):
    return pl.pallas_call(
        _comb_relu_mm_body,
        grid=(N // _MM_BLOCK,),
        in_specs=[
            pl.BlockSpec((1, _MM_BLOCK, D), lambda i: (0, i, 0)),
            pl.BlockSpec((1, _MM_BLOCK, D), lambda i: (1, i, 0)),
            pl.BlockSpec((1, D), lambda i: (0, 0)),
            pl.BlockSpec((D, D), lambda i: (0, 0)),
        ],
        out_specs=pl.BlockSpec((_MM_BLOCK, D), lambda i: (i, 0)),
        out_shape=jax.ShapeDtypeStruct((N, D), jnp.float32),
    )(p, p, b, w)


def _comb_body(p_ref0, p_ref1, b_ref, o_ref):
    o_ref[...] = p_ref0[0] + p_ref1[0] + b_ref[...]


def _tc_combine(p, b):
    return pl.pallas_call(
        _comb_body,
        grid=(N // _MM_BLOCK,),
        in_specs=[
            pl.BlockSpec((1, _MM_BLOCK, D), lambda i: (0, i, 0)),
            pl.BlockSpec((1, _MM_BLOCK, D), lambda i: (1, i, 0)),
            pl.BlockSpec((1, D), lambda i: (0, 0)),
        ],
        out_specs=pl.BlockSpec((_MM_BLOCK, D), lambda i: (i, 0)),
        out_shape=jax.ShapeDtypeStruct((N, D), jnp.float32),
    )(p, p, b)


@functools.partial(
    pl.kernel,
    out_type=jax.ShapeDtypeStruct((NC, N_PAD, D), jnp.float32),
    mesh=plsc.VectorSubcoreMesh(core_axis_name="c", subcore_axis_name="s"),
    scratch_types=[
        pltpu.VMEM((2, NBUF, K), jnp.int32),
        pltpu.VMEM((2, NBUF, K), jnp.int32),
        pltpu.VMEM((NBUF, K, D), jnp.float32),
        pltpu.VMEM_SHARED((N_PAD, D), jnp.float32),
        [pltpu.SemaphoreType.DMA] * NBUF,
        [pltpu.SemaphoreType.DMA] * NBUF,
        [pltpu.SemaphoreType.DMA] * 2,
    ],
)
def _sc_segment_sum(h_hbm, src_hbm, dst_hbm, zeros_hbm, out_hbm,
                    src_sl, dst_sl, rows_v, accum, gsems, ssems, slsems):
    c = lax.axis_index("c")
    s = lax.axis_index("s")
    wid = s * NC + c

    pltpu.sync_copy(zeros_hbm.at[pl.ds(s * ROWS_PT, ROWS_PT)],
                    accum.at[pl.ds(s * ROWS_PT, ROWS_PT)])
    plsc.subcore_barrier()

    def gather(hb, b):
        return pltpu.async_copy(h_hbm.at[src_sl.at[hb, b]], rows_v.at[b],
                                gsems[b])

    def scatter(hb, b):
        return pltpu.async_copy(rows_v.at[b], accum.at[dst_sl.at[hb, b]],
                                ssems[b], add=True)

    def load_slabs(g, hb, sync=False):
        if sync:
            pltpu.sync_copy(src_hbm.at[wid, g], src_sl.at[hb])
            pltpu.sync_copy(dst_hbm.at[wid, g], dst_sl.at[hb])
        else:
            pltpu.async_copy(src_hbm.at[wid, g], src_sl.at[hb], slsems[hb])
            pltpu.async_copy(dst_hbm.at[wid, g], dst_sl.at[hb], slsems[hb])

    def wait_slabs(g, hb):
        pltpu.make_async_copy(src_hbm.at[wid, g], src_sl.at[hb],
                              slsems[hb]).wait()
        pltpu.make_async_copy(dst_hbm.at[wid, g], dst_sl.at[hb],
                              slsems[hb]).wait()

    load_slabs(0, 0, sync=True)
    for b in range(NBUF):
        gather(0, b)
    load_slabs(1, 1)

    def pair_body(t, _):
        for hb in range(2):
            g = 2 * t + hb
            nhb = 1 - hb
            for b in range(NBUF):
                pltpu.make_async_copy(h_hbm.at[src_sl.at[hb, b]],
                                      rows_v.at[b], gsems[b]).wait()
            for b in range(NBUF):
                scatter(hb, b)
            @pl.when(g < GROUPS - 1)
            def _():
                wait_slabs(g + 1, nhb)
            for b in range(NBUF):
                pltpu.make_async_copy(rows_v.at[b],
                                      accum.at[dst_sl.at[hb, b]],
                                      ssems[b]).wait()

                @pl.when(g < GROUPS - 1)
                def _():
                    gather(nhb, b)
            @pl.when(g < GROUPS - 2)
            def _():
                load_slabs(g + 2, hb)
        return 0

    lax.fori_loop(0, GROUPS // 2, pair_body, 0)

    plsc.subcore_barrier()
    pltpu.sync_copy(accum.at[pl.ds(s * ROWS_PT, ROWS_PT)],
                    out_hbm.at[c, pl.ds(s * ROWS_PT, ROWS_PT)])


def kernel(x, edge_index, W1, b1, W2, b2):
    pad = E_PAD - E
    dst = jnp.concatenate(
        [edge_index[0], jnp.full((pad,), N, jnp.int32)]).reshape(
            NW, GROUPS, NBUF, K)
    src = jnp.concatenate(
        [edge_index[1], jnp.zeros((pad,), jnp.int32)]).reshape(
            NW, GROUPS, NBUF, K)
    zeros = jnp.zeros((N_PAD, D), jnp.float32)
    b1r = b1.reshape(1, D)
    b2r = b2.reshape(1, D)

    h = _tc_matmul(x, W1)
    p = _sc_segment_sum(h, src, dst, zeros)
    h = _tc_combine_relu_matmul(p, b1r, W2)
    q = _sc_segment_sum(h, src, dst, zeros)
    return _tc_combine(q, b2r)


# speedup vs baseline: 2.8848x; 2.8848x over previous
import functools

import jax
import jax.numpy as jnp
from jax import lax
from jax.experimental import pallas as pl
from jax.experimental.pallas import tpu as pltpu
from jax.experimental.pallas import tpu_sc as plsc

N = 10000
D = 128
E = 320000

NC = 2
NS = 16
NW = NC * NS

K = 100
EPT = E // NW
CHUNKS = EPT // K
N_PAD = 10112
ROWS_PT = N_PAD // NS

_MM_BLOCK = 1000


def _mm_body(x_ref, w_ref, o_ref):
    o_ref[...] = jnp.dot(x_ref[...], w_ref[...],
                         preferred_element_type=jnp.float32)


def _tc_matmul(x, w):
    return pl.pallas_call(
        _mm_body,
        grid=(N // _MM_BLOCK,),
        in_specs=[
            pl.BlockSpec((_MM_BLOCK, D), lambda i: (i, 0)),
            pl.BlockSpec((D, D), lambda i: (0, 0)),
        ],
        out_specs=pl.BlockSpec((_MM_BLOCK, D), lambda i: (i, 0)),
        out_shape=jax.ShapeDtypeStruct((N, D), jnp.float32),
    )(x, w)


def _comb_relu_mm_body(p_ref0, p_ref1, b_ref, w_ref, o_ref):
    h = jnp.maximum(p_ref0[0] + p_ref1[0] + b_ref[...], 0.0)
    o_ref[...] = jnp.dot(h, w_ref[...], preferred_element_type=jnp.float32)


def _tc_combine_relu_matmul(p, b, w):
    return pl.pallas_call(
        _comb_relu_mm_body,
        grid=(N // _MM_BLOCK,),
        in_specs=[
            pl.BlockSpec((1, _MM_BLOCK, D), lambda i: (0, i, 0)),
            pl.BlockSpec((1, _MM_BLOCK, D), lambda i: (1, i, 0)),
            pl.BlockSpec((1, D), lambda i: (0, 0)),
            pl.BlockSpec((D, D), lambda i: (0, 0)),
        ],
        out_specs=pl.BlockSpec((_MM_BLOCK, D), lambda i: (i, 0)),
        out_shape=jax.ShapeDtypeStruct((N, D), jnp.float32),
    )(p, p, b, w)


def _comb_body(p_ref0, p_ref1, b_ref, o_ref):
    o_ref[...] = p_ref0[0] + p_ref1[0] + b_ref[...]


def _tc_combine(p, b):
    return pl.pallas_call(
        _comb_body,
        grid=(N // _MM_BLOCK,),
        in_specs=[
            pl.BlockSpec((1, _MM_BLOCK, D), lambda i: (0, i, 0)),
            pl.BlockSpec((1, _MM_BLOCK, D), lambda i: (1, i, 0)),
            pl.BlockSpec((1, D), lambda i: (0, 0)),
        ],
        out_specs=pl.BlockSpec((_MM_BLOCK, D), lambda i: (i, 0)),
        out_shape=jax.ShapeDtypeStruct((N, D), jnp.float32),
    )(p, p, b)


@functools.partial(
    pl.kernel,
    out_type=jax.ShapeDtypeStruct((NC, N_PAD, D), jnp.float32),
    mesh=plsc.VectorSubcoreMesh(core_axis_name="c", subcore_axis_name="s"),
    scratch_types=[
        pltpu.VMEM((2, 2, K), jnp.int32),
        pltpu.VMEM((2, K, D), jnp.float32),
        pltpu.VMEM_SHARED((N_PAD, D), jnp.float32),
        [pltpu.SemaphoreType.DMA] * 2,
        [pltpu.SemaphoreType.DMA] * 2,
    ],
)
def _sc_segment_sum(h_hbm, eidx_hbm, zeros_hbm, out_hbm,
                    slab, rows_v, accum, gsems, slsems):
    c = lax.axis_index("c")
    s = lax.axis_index("s")
    wid = s * NC + c

    pltpu.sync_copy(zeros_hbm.at[pl.ds(s * ROWS_PT, ROWS_PT)],
                    accum.at[pl.ds(s * ROWS_PT, ROWS_PT)])
    plsc.subcore_barrier()

    def gather(p):
        pltpu.async_copy(h_hbm.at[slab.at[p, 0]], rows_v.at[p], gsems[p])

    def wait_gather(p):
        pltpu.make_async_copy(h_hbm.at[slab.at[p, 0]], rows_v.at[p],
                              gsems[p]).wait()

    def scatter(p):
        pltpu.sync_copy(rows_v.at[p], accum.at[slab.at[p, 1]], add=True)

    def load_slab(j, p):
        pltpu.async_copy(eidx_hbm.at[wid, j], slab.at[p], slsems[p])

    def wait_slab(j, p):
        pltpu.make_async_copy(eidx_hbm.at[wid, j], slab.at[p],
                              slsems[p]).wait()

    pltpu.sync_copy(eidx_hbm.at[wid, 0], slab.at[0])
    gather(0)
    load_slab(1, 1)

    def pair_body(t, _):
        for p in range(2):
            j = 2 * t + p
            np_ = 1 - p

            @pl.when(j + 1 < CHUNKS)
            def _():
                wait_slab(j + 1, np_)
                gather(np_)

            wait_gather(p)
            scatter(p)

            @pl.when(j + 2 < CHUNKS)
            def _():
                load_slab(j + 2, p)
        return 0

    lax.fori_loop(0, CHUNKS // 2, pair_body, 0)

    plsc.subcore_barrier()
    pltpu.sync_copy(accum.at[pl.ds(s * ROWS_PT, ROWS_PT)],
                    out_hbm.at[c, pl.ds(s * ROWS_PT, ROWS_PT)])


def kernel(x, edge_index, W1, b1, W2, b2):
    eidx = jnp.stack([edge_index[1].reshape(NW, CHUNKS, K),
                      edge_index[0].reshape(NW, CHUNKS, K)], axis=2)
    zeros = jnp.zeros((N_PAD, D), jnp.float32)
    b1r = b1.reshape(1, D)
    b2r = b2.reshape(1, D)

    h = _tc_matmul(x, W1)
    p = _sc_segment_sum(h, eidx, zeros)
    h = _tc_combine_relu_matmul(p, b1r, W2)
    q = _sc_segment_sum(h, eidx, zeros)
    return _tc_combine(q, b2r)


# speedup vs baseline: 3.2708x; 1.1338x over previous
import functools

import jax
import jax.numpy as jnp
from jax import lax
from jax.experimental import pallas as pl
from jax.experimental.pallas import tpu as pltpu
from jax.experimental.pallas import tpu_sc as plsc

N = 10000
D = 128
E = 320000

NC = 2
NS = 16
NW = NC * NS

K = 100
EPT = E // NW
CHUNKS = EPT // K
N_PAD = 10112
ROWS_PT = N_PAD // NS

_MM_BLOCK = 1000


def _mm_body(x_ref, w_ref, o_ref):
    o_ref[...] = jnp.dot(x_ref[...], w_ref[...],
                         preferred_element_type=jnp.float32)


def _tc_matmul(x, w):
    return pl.pallas_call(
        _mm_body,
        grid=(N // _MM_BLOCK,),
        in_specs=[
            pl.BlockSpec((_MM_BLOCK, D), lambda i: (i, 0)),
            pl.BlockSpec((D, D), lambda i: (0, 0)),
        ],
        out_specs=pl.BlockSpec((_MM_BLOCK, D), lambda i: (i, 0)),
        out_shape=jax.ShapeDtypeStruct((N, D), jnp.float32),
    )(x, w)


def _comb_relu_mm_body(p_ref0, p_ref1, b_ref, w_ref, o_ref):
    h = jnp.maximum(p_ref0[0] + p_ref1[0] + b_ref[...], 0.0)
    o_ref[...] = jnp.dot(h, w_ref[...], preferred_element_type=jnp.float32)


def _tc_combine_relu_matmul(p, b, w):
    return pl.pallas_call(
        _comb_relu_mm_body,
        grid=(N // _MM_BLOCK,),
        in_specs=[
            pl.BlockSpec((1, _MM_BLOCK, D), lambda i: (0, i, 0)),
            pl.BlockSpec((1, _MM_BLOCK, D), lambda i: (1, i, 0)),
            pl.BlockSpec((1, D), lambda i: (0, 0)),
            pl.BlockSpec((D, D), lambda i: (0, 0)),
        ],
        out_specs=pl.BlockSpec((_MM_BLOCK, D), lambda i: (i, 0)),
        out_shape=jax.ShapeDtypeStruct((N, D), jnp.float32),
    )(p, p, b, w)


def _comb_body(p_ref0, p_ref1, b_ref, o_ref):
    o_ref[...] = p_ref0[0] + p_ref1[0] + b_ref[...]


def _tc_combine(p, b):
    return pl.pallas_call(
        _comb_body,
        grid=(N // _MM_BLOCK,),
        in_specs=[
            pl.BlockSpec((1, _MM_BLOCK, D), lambda i: (0, i, 0)),
            pl.BlockSpec((1, _MM_BLOCK, D), lambda i: (1, i, 0)),
            pl.BlockSpec((1, D), lambda i: (0, 0)),
        ],
        out_specs=pl.BlockSpec((_MM_BLOCK, D), lambda i: (i, 0)),
        out_shape=jax.ShapeDtypeStruct((N, D), jnp.float32),
    )(p, p, b)


@functools.partial(
    pl.kernel,
    out_type=jax.ShapeDtypeStruct((NC, N_PAD, D), jnp.float32),
    mesh=plsc.VectorSubcoreMesh(core_axis_name="c", subcore_axis_name="s"),
    scratch_types=[
        pltpu.VMEM((4, 2, K), jnp.int32),
        pltpu.VMEM((2, K, D), jnp.float32),
        pltpu.VMEM_SHARED((N_PAD, D), jnp.float32),
        [pltpu.SemaphoreType.DMA] * 2,
        [pltpu.SemaphoreType.DMA] * 2,
        [pltpu.SemaphoreType.DMA] * 4,
    ],
)
def _sc_segment_sum(h_hbm, eidx_hbm, zeros_hbm, out_hbm,
                    slab, rows_v, accum, gsems, ssems, slsems):
    c = lax.axis_index("c")
    s = lax.axis_index("s")
    wid = s * NC + c

    pltpu.sync_copy(zeros_hbm.at[pl.ds(s * ROWS_PT, ROWS_PT)],
                    accum.at[pl.ds(s * ROWS_PT, ROWS_PT)])
    plsc.subcore_barrier()

    def gather(sl, p):
        pltpu.async_copy(h_hbm.at[slab.at[sl, 0]], rows_v.at[p], gsems[p])

    def wait_gather(sl, p):
        pltpu.make_async_copy(h_hbm.at[slab.at[sl, 0]], rows_v.at[p],
                              gsems[p]).wait()

    def scatter(sl, p):
        pltpu.async_copy(rows_v.at[p], accum.at[slab.at[sl, 1]], ssems[p],
                         add=True)

    def wait_scatter(sl, p):
        pltpu.make_async_copy(rows_v.at[p], accum.at[slab.at[sl, 1]],
                              ssems[p]).wait()

    def load_slab(j, sl):
        pltpu.async_copy(eidx_hbm.at[wid, j], slab.at[sl], slsems[sl])

    def wait_slab(j, sl):
        pltpu.make_async_copy(eidx_hbm.at[wid, j], slab.at[sl],
                              slsems[sl]).wait()

    pltpu.sync_copy(eidx_hbm.at[wid, 0], slab.at[0])
    gather(0, 0)
    load_slab(1, 1)

    def quad_body(t, _):
        for q in range(4):
            j = 4 * t + q
            p = q % 2
            np_ = 1 - p

            if q == 0:
                @pl.when(t > 0)
                def _():
                    wait_scatter((q - 1) % 4, np_)
            else:
                wait_scatter(q - 1, np_)

            @pl.when(j + 1 < CHUNKS)
            def _():
                wait_slab(j + 1, (q + 1) % 4)
                gather((q + 1) % 4, np_)

            wait_gather(q, p)
            scatter(q, p)

            @pl.when(j + 2 < CHUNKS)
            def _():
                load_slab(j + 2, (q + 2) % 4)
        return 0

    lax.fori_loop(0, CHUNKS // 4, quad_body, 0)

    wait_scatter((CHUNKS - 1) % 4, (CHUNKS - 1) % 2)

    plsc.subcore_barrier()
    pltpu.sync_copy(accum.at[pl.ds(s * ROWS_PT, ROWS_PT)],
                    out_hbm.at[c, pl.ds(s * ROWS_PT, ROWS_PT)])


def kernel(x, edge_index, W1, b1, W2, b2):
    eidx = jnp.stack([edge_index[1].reshape(NW, CHUNKS, K),
                      edge_index[0].reshape(NW, CHUNKS, K)], axis=2)
    zeros = jnp.zeros((N_PAD, D), jnp.float32)
    b1r = b1.reshape(1, D)
    b2r = b2.reshape(1, D)

    h = _tc_matmul(x, W1)
    p = _sc_segment_sum(h, eidx, zeros)
    h = _tc_combine_relu_matmul(p, b1r, W2)
    q = _sc_segment_sum(h, eidx, zeros)
    return _tc_combine(q, b2r)


# speedup vs baseline: 3.3742x; 1.0316x over previous
import functools

import jax
import jax.numpy as jnp
from jax import lax
from jax.experimental import pallas as pl
from jax.experimental.pallas import tpu as pltpu
from jax.experimental.pallas import tpu_sc as plsc

N = 10000
D = 128
E = 320000

NC = 2
NS = 16
NW = NC * NS

K = 100
EPT = E // NW
CHUNKS = EPT // K
N_PAD = 10112
ROWS_PT = N_PAD // NS

_MM_BLOCK = 1000


def _fused_mm_body(p_ref0, p_ref1, b_ref, w1_ref, w2_ref, o_ref):
    h = jnp.maximum(
        jnp.dot(p_ref0[0] + p_ref1[0], w1_ref[...],
                preferred_element_type=jnp.float32) + b_ref[...], 0.0)
    o_ref[...] = jnp.dot(h, w2_ref[...], preferred_element_type=jnp.float32)


def _tc_fused_mms(p, b, w1, w2):
    return pl.pallas_call(
        _fused_mm_body,
        grid=(N // _MM_BLOCK,),
        in_specs=[
            pl.BlockSpec((1, _MM_BLOCK, D), lambda i: (0, i, 0)),
            pl.BlockSpec((1, _MM_BLOCK, D), lambda i: (1, i, 0)),
            pl.BlockSpec((1, D), lambda i: (0, 0)),
            pl.BlockSpec((D, D), lambda i: (0, 0)),
            pl.BlockSpec((D, D), lambda i: (0, 0)),
        ],
        out_specs=pl.BlockSpec((_MM_BLOCK, D), lambda i: (i, 0)),
        out_shape=jax.ShapeDtypeStruct((N, D), jnp.float32),
    )(p, p, b, w1, w2)


def _comb_body(p_ref0, p_ref1, b_ref, o_ref):
    o_ref[...] = p_ref0[0] + p_ref1[0] + b_ref[...]


def _tc_combine(p, b):
    return pl.pallas_call(
        _comb_body,
        grid=(N // _MM_BLOCK,),
        in_specs=[
            pl.BlockSpec((1, _MM_BLOCK, D), lambda i: (0, i, 0)),
            pl.BlockSpec((1, _MM_BLOCK, D), lambda i: (1, i, 0)),
            pl.BlockSpec((1, D), lambda i: (0, 0)),
        ],
        out_specs=pl.BlockSpec((_MM_BLOCK, D), lambda i: (i, 0)),
        out_shape=jax.ShapeDtypeStruct((N, D), jnp.float32),
    )(p, p, b)


@functools.partial(
    pl.kernel,
    out_type=jax.ShapeDtypeStruct((NC, N_PAD, D), jnp.float32),
    mesh=plsc.VectorSubcoreMesh(core_axis_name="c", subcore_axis_name="s"),
    scratch_types=[
        pltpu.VMEM((4, 2, K), jnp.int32),
        pltpu.VMEM((2, K, D), jnp.float32),
        pltpu.VMEM_SHARED((N_PAD, D), jnp.float32),
        [pltpu.SemaphoreType.DMA] * 2,
        [pltpu.SemaphoreType.DMA] * 2,
        [pltpu.SemaphoreType.DMA] * 4,
    ],
)
def _sc_segment_sum(h_hbm, eidx_hbm, zeros_hbm, out_hbm,
                    slab, rows_v, accum, gsems, ssems, slsems):
    c = lax.axis_index("c")
    s = lax.axis_index("s")
    wid = s * NC + c

    pltpu.sync_copy(zeros_hbm.at[pl.ds(s * ROWS_PT, ROWS_PT)],
                    accum.at[pl.ds(s * ROWS_PT, ROWS_PT)])
    plsc.subcore_barrier()

    def gather(sl, p):
        pltpu.async_copy(h_hbm.at[slab.at[sl, 0]], rows_v.at[p], gsems[p])

    def wait_gather(sl, p):
        pltpu.make_async_copy(h_hbm.at[slab.at[sl, 0]], rows_v.at[p],
                              gsems[p]).wait()

    def scatter(sl, p):
        pltpu.async_copy(rows_v.at[p], accum.at[slab.at[sl, 1]], ssems[p],
                         add=True)

    def wait_scatter(sl, p):
        pltpu.make_async_copy(rows_v.at[p], accum.at[slab.at[sl, 1]],
                              ssems[p]).wait()

    def load_slab(j, sl):
        pltpu.async_copy(eidx_hbm.at[wid, j], slab.at[sl], slsems[sl])

    def wait_slab(j, sl):
        pltpu.make_async_copy(eidx_hbm.at[wid, j], slab.at[sl],
                              slsems[sl]).wait()

    pltpu.sync_copy(eidx_hbm.at[wid, 0], slab.at[0])
    gather(0, 0)
    load_slab(1, 1)

    def quad_body(t, _):
        for q in range(4):
            j = 4 * t + q
            p = q % 2
            np_ = 1 - p

            if q == 0:
                @pl.when(t > 0)
                def _():
                    wait_scatter((q - 1) % 4, np_)
            else:
                wait_scatter(q - 1, np_)

            @pl.when(j + 1 < CHUNKS)
            def _():
                wait_slab(j + 1, (q + 1) % 4)
                gather((q + 1) % 4, np_)

            wait_gather(q, p)
            scatter(q, p)

            @pl.when(j + 2 < CHUNKS)
            def _():
                load_slab(j + 2, (q + 2) % 4)
        return 0

    lax.fori_loop(0, CHUNKS // 4, quad_body, 0)

    wait_scatter((CHUNKS - 1) % 4, (CHUNKS - 1) % 2)

    plsc.subcore_barrier()
    pltpu.sync_copy(accum.at[pl.ds(s * ROWS_PT, ROWS_PT)],
                    out_hbm.at[c, pl.ds(s * ROWS_PT, ROWS_PT)])


def kernel(x, edge_index, W1, b1, W2, b2):
    eidx = jnp.stack([edge_index[1].reshape(NW, CHUNKS, K),
                      edge_index[0].reshape(NW, CHUNKS, K)], axis=2)
    zeros = jnp.zeros((N_PAD, D), jnp.float32)
    b1r = b1.reshape(1, D)
    b2r = b2.reshape(1, D)

    p = _sc_segment_sum(x, eidx, zeros)
    h = _tc_fused_mms(p, b1r, W1, W2)
    q = _sc_segment_sum(h, eidx, zeros)
    return _tc_combine(q, b2r)


# speedup vs baseline: 3.4070x; 1.0097x over previous
import functools

import jax
import jax.numpy as jnp
from jax import lax
from jax.experimental import pallas as pl
from jax.experimental.pallas import tpu as pltpu
from jax.experimental.pallas import tpu_sc as plsc

N = 10000
D = 128
E = 320000

NC = 2
NS = 16
NW = NC * NS

K = 100
EPT = E // NW
CHUNKS = EPT // K
N_PAD = 10112
ROWS_PT = N_PAD // NS

_MM_BLOCK = 1000


def _fused_mm_body(p_ref0, p_ref1, b_ref, w1_ref, w2_ref, o_ref):
    h = jnp.maximum(
        jnp.dot(p_ref0[0] + p_ref1[0], w1_ref[...],
                preferred_element_type=jnp.float32) + b_ref[...], 0.0)
    o_ref[...] = jnp.dot(h, w2_ref[...], preferred_element_type=jnp.float32)


def _tc_fused_mms(p, b, w1, w2):
    return pl.pallas_call(
        _fused_mm_body,
        grid=(N // _MM_BLOCK,),
        in_specs=[
            pl.BlockSpec((1, _MM_BLOCK, D), lambda i: (0, i, 0)),
            pl.BlockSpec((1, _MM_BLOCK, D), lambda i: (1, i, 0)),
            pl.BlockSpec((1, D), lambda i: (0, 0)),
            pl.BlockSpec((D, D), lambda i: (0, 0)),
            pl.BlockSpec((D, D), lambda i: (0, 0)),
        ],
        out_specs=pl.BlockSpec((_MM_BLOCK, D), lambda i: (i, 0)),
        out_shape=jax.ShapeDtypeStruct((N, D), jnp.float32),
    )(p, p, b, w1, w2)


def _comb_body(p_ref0, p_ref1, b_ref, o_ref):
    o_ref[...] = p_ref0[0] + p_ref1[0] + b_ref[...]


def _tc_combine(p, b):
    return pl.pallas_call(
        _comb_body,
        grid=(N // _MM_BLOCK,),
        in_specs=[
            pl.BlockSpec((1, _MM_BLOCK, D), lambda i: (0, i, 0)),
            pl.BlockSpec((1, _MM_BLOCK, D), lambda i: (1, i, 0)),
            pl.BlockSpec((1, D), lambda i: (0, 0)),
        ],
        out_specs=pl.BlockSpec((_MM_BLOCK, D), lambda i: (i, 0)),
        out_shape=jax.ShapeDtypeStruct((N, D), jnp.float32),
    )(p, p, b)


@functools.partial(
    pl.kernel,
    out_type=jax.ShapeDtypeStruct((NC, N_PAD, D), jnp.float32),
    mesh=plsc.VectorSubcoreMesh(core_axis_name="c", subcore_axis_name="s"),
    scratch_types=[
        pltpu.VMEM((4, 2, K), jnp.int32),
        pltpu.VMEM((2, K, D), jnp.float32),
        pltpu.VMEM_SHARED((N_PAD, D), jnp.float32),
        [pltpu.SemaphoreType.DMA] * 2,
        [pltpu.SemaphoreType.DMA] * 2,
        [pltpu.SemaphoreType.DMA] * 4,
        pltpu.SemaphoreType.DMA,
    ],
)
def _sc_segment_sum(h_hbm, eidx_hbm, zeros_hbm, out_hbm,
                    slab, rows_v, accum, gsems, ssems, slsems, zsem):
    c = lax.axis_index("c")
    s = lax.axis_index("s")
    wid = s * NC + c

    zdesc = pltpu.async_copy(zeros_hbm.at[pl.ds(s * ROWS_PT, ROWS_PT)],
                             accum.at[pl.ds(s * ROWS_PT, ROWS_PT)], zsem)

    def gather(sl, p):
        pltpu.async_copy(h_hbm.at[slab.at[sl, 0]], rows_v.at[p], gsems[p])

    def wait_gather(sl, p):
        pltpu.make_async_copy(h_hbm.at[slab.at[sl, 0]], rows_v.at[p],
                              gsems[p]).wait()

    def scatter(sl, p):
        pltpu.async_copy(rows_v.at[p], accum.at[slab.at[sl, 1]], ssems[p],
                         add=True)

    def wait_scatter(sl, p):
        pltpu.make_async_copy(rows_v.at[p], accum.at[slab.at[sl, 1]],
                              ssems[p]).wait()

    def load_slab(j, sl):
        pltpu.async_copy(eidx_hbm.at[wid, j], slab.at[sl], slsems[sl])

    def wait_slab(j, sl):
        pltpu.make_async_copy(eidx_hbm.at[wid, j], slab.at[sl],
                              slsems[sl]).wait()

    pltpu.sync_copy(eidx_hbm.at[wid, 0], slab.at[0])
    gather(0, 0)
    load_slab(1, 1)
    zdesc.wait()
    plsc.subcore_barrier()

    def quad_body(t, _):
        for q in range(4):
            j = 4 * t + q
            p = q % 2
            np_ = 1 - p

            if q == 0:
                @pl.when(t > 0)
                def _():
                    wait_scatter((q - 1) % 4, np_)
            else:
                wait_scatter(q - 1, np_)

            @pl.when(j + 1 < CHUNKS)
            def _():
                wait_slab(j + 1, (q + 1) % 4)
                gather((q + 1) % 4, np_)

            wait_gather(q, p)
            scatter(q, p)

            @pl.when(j + 2 < CHUNKS)
            def _():
                load_slab(j + 2, (q + 2) % 4)
        return 0

    lax.fori_loop(0, CHUNKS // 4, quad_body, 0)

    wait_scatter((CHUNKS - 1) % 4, (CHUNKS - 1) % 2)

    plsc.subcore_barrier()
    pltpu.sync_copy(accum.at[pl.ds(s * ROWS_PT, ROWS_PT)],
                    out_hbm.at[c, pl.ds(s * ROWS_PT, ROWS_PT)])


def kernel(x, edge_index, W1, b1, W2, b2):
    eidx = jnp.stack([edge_index[1].reshape(NW, CHUNKS, K),
                      edge_index[0].reshape(NW, CHUNKS, K)], axis=2)
    zeros = jnp.zeros((N_PAD, D), jnp.float32)
    b1r = b1.reshape(1, D)
    b2r = b2.reshape(1, D)

    p = _sc_segment_sum(x, eidx, zeros)
    h = _tc_fused_mms(p, b1r, W1, W2)
    q = _sc_segment_sum(h, eidx, zeros)
    return _tc_combine(q, b2r)
